# baseline (speedup 1.0000x reference)
import functools

import jax
import jax.numpy as jnp
from jax import lax
from jax.experimental import pallas as pl
from jax.experimental.pallas import tpu as pltpu
from jax.experimental.pallas import tpu_sc as plsc

N = 10000
V = 2230
D = 128
G = 32
C = 2
GX = 64
EC, ED, EK = 160000, 120000, 40000
E = 2 * (EC + ED + EK)
NUM_ET = 6

NC, NS = 2, 16
NW = NC * NS
K = 128
SB = 4
GROUPS = -(-E // (NW * K * SB))
CHP = GROUPS * SB
E_PAD = NW * CHP * K
EMB_CH = 3
IDS_PAD = NW * EMB_CH * K
NPAD = 10112
RPS = NPAD // NS

BS = 1000
NB = N // BS

f32 = jnp.float32
i32 = jnp.int32

_SC_MESH = dict(core_axis_name="c", subcore_axis_name="s")


def _emb_body(emb_hbm, ids_hbm, out_hbm, idx_v, rows_v, sem):
    c = lax.axis_index("c")
    s = lax.axis_index("s")
    w = s * NC + c
    for j in range(EMB_CH):
        base = (w * EMB_CH + j) * K
        pltpu.sync_copy(ids_hbm.at[pl.ds(base, K)], idx_v)
        pltpu.async_copy(emb_hbm.at[idx_v], rows_v, sem).wait()
        pltpu.sync_copy(rows_v, out_hbm.at[pl.ds(base, K)])


def _emb_gather(embed, ids_pad):
    fn = functools.partial(
        pl.kernel,
        out_type=jax.ShapeDtypeStruct((IDS_PAD, D), f32),
        mesh=plsc.VectorSubcoreMesh(**_SC_MESH),
        scratch_types=[
            pltpu.VMEM((K,), i32),
            pltpu.VMEM((K, D), f32),
            pltpu.SemaphoreType.DMA,
        ],
    )(_emb_body)
    return fn(embed, ids_pad)


def _agg_body(hw_hbm, idx_hbm, zeros_hbm, out_hbm,
              idxv, rows, agg_sh, sem):
    c = lax.axis_index("c")
    s = lax.axis_index("s")
    w = s * NC + c
    pltpu.sync_copy(zeros_hbm, agg_sh.at[pl.ds(s * RPS, RPS)])
    plsc.subcore_barrier()

    def group(g, carry):
        pltpu.sync_copy(idx_hbm.at[w, g], idxv)
        for b in range(SB):
            pltpu.async_copy(hw_hbm.at[idxv.at[b, 0]], rows, sem).wait()
            pltpu.sync_copy(rows, agg_sh.at[idxv.at[b, 1]], add=True)
        return carry

    lax.fori_loop(0, GROUPS, group, 0)
    plsc.subcore_barrier()
    pltpu.sync_copy(agg_sh.at[pl.ds(s * RPS, RPS)],
                    out_hbm.at[pl.ds(c * NPAD + s * RPS, RPS)])


def _agg_scatter(hw_flat, idx_all, zeros_rows):
    fn = functools.partial(
        pl.kernel,
        out_type=jax.ShapeDtypeStruct((NC * NPAD, D), f32),
        mesh=plsc.VectorSubcoreMesh(**_SC_MESH),
        scratch_types=[
            pltpu.VMEM((SB, 2, K), i32),
            pltpu.VMEM((K, D), f32),
            pltpu.VMEM_SHARED((NPAD, D), f32),
            pltpu.SemaphoreType.DMA,
        ],
    )(_agg_body)
    return fn(hw_flat, idx_all, zeros_rows)


def _hw_body(h_ref, w_ref, b_ref, out_ref):
    out_ref[0] = (jnp.dot(h_ref[...], w_ref[0], preferred_element_type=f32)
                  + b_ref[0])


def _hw_matmul(h, Wl, bl):
    return pl.pallas_call(
        _hw_body,
        grid=(NUM_ET, NB),
        in_specs=[
            pl.BlockSpec((BS, D), lambda e, i: (i, 0)),
            pl.BlockSpec((1, D, D), lambda e, i: (e, 0, 0)),
            pl.BlockSpec((1, 1, D), lambda e, i: (e, 0, 0)),
        ],
        out_specs=pl.BlockSpec((1, BS, D), lambda e, i: (e, i, 0)),
        out_shape=jax.ShapeDtypeStruct((NUM_ET, N, D), f32),
    )(h, Wl, bl)


def _gru_body(parts_ref, h_ref, gw_ref, gu_ref, gb_ref, out_ref):
    agg = parts_ref[0] + parts_ref[1]
    h = h_ref[...]
    dot = lambda a, b: jnp.dot(a, b, preferred_element_type=f32)
    z = jax.nn.sigmoid(dot(agg, gw_ref[0]) + dot(h, gu_ref[0]) + gb_ref[0])
    r = jax.nn.sigmoid(dot(agg, gw_ref[1]) + dot(h, gu_ref[1]) + gb_ref[1])
    hh = jnp.tanh(dot(agg, gw_ref[2]) + dot(r * h, gu_ref[2]) + gb_ref[2])
    out_ref[...] = (1.0 - z) * h + z * hh


def _gru_apply(parts3, h, gW, gU, gb):
    return pl.pallas_call(
        _gru_body,
        grid=(NB,),
        in_specs=[
            pl.BlockSpec((NC, BS, D), lambda i: (0, i, 0)),
            pl.BlockSpec((BS, D), lambda i: (i, 0)),
            pl.BlockSpec((3, D, D), lambda i: (0, 0, 0)),
            pl.BlockSpec((3, D, D), lambda i: (0, 0, 0)),
            pl.BlockSpec((3, D), lambda i: (0, 0)),
        ],
        out_specs=pl.BlockSpec((BS, D), lambda i: (i, 0)),
        out_shape=jax.ShapeDtypeStruct((N, D), f32),
    )(parts3, h, gW, gU, gb)


def _readout_body(h_ref, h0_ref, gid_ref, wf_ref, bf_ref, wg_ref, bg_ref,
                  aux_ref, w1_ref, b1_ref, w2_ref, b2_ref, out_ref):
    h = h_ref[...]
    h0 = h0_ref[...]
    dot = lambda a, b: jnp.dot(a, b, preferred_element_type=f32)
    gate = jax.nn.sigmoid(dot(h, wf_ref[:D]) + dot(h0, wf_ref[D:]) + bf_ref[0])
    val = dot(h, wg_ref[...]) + bg_ref[0]
    gv = gate * val
    onehot = (lax.broadcasted_iota(i32, (G, N), 0) == gid_ref[...]).astype(f32)
    feats = dot(onehot, gv)
    aux = aux_ref[...]
    x = (feats[:, 0:1] * w1_ref[0:1, :] + feats[:, 1:2] * w1_ref[1:2, :]
         + aux[:, 0:1] * w1_ref[2:3, :] + aux[:, 1:2] * w1_ref[3:4, :]
         + b1_ref[...])
    x = jnp.maximum(x, 0.0)
    out_ref[...] = dot(x, w2_ref[...]) + b2_ref[...]


def _readout(h, h0, gid2d, Wf, bf2, Wg, bg2, aux, W1, b12, W2, b22):
    return pl.pallas_call(
        _readout_body,
        out_shape=jax.ShapeDtypeStruct((G, C), f32),
    )(h, h0, gid2d, Wf, bf2, Wg, bg2, aux, W1, b12, W2, b22)


def kernel(node_vocab_ids, control_edge_index, data_edge_index, call_edge_index,
           graph_nodes_list, wgsize_log1p, transfer_bytes_log1p,
           embed, mp1_W, mp1_b, mp1_gru_W, mp1_gru_U, mp1_gru_b,
           mp2_W, mp2_b, mp2_gru_W, mp2_gru_U, mp2_gru_b,
           Wf, bf, Wg, bg, W1, b1, W2, b2):
    ids_pad = jnp.concatenate(
        [node_vocab_ids.astype(i32), jnp.zeros((IDS_PAD - N,), i32)])
    src_list = [control_edge_index[0], data_edge_index[0], call_edge_index[0],
                control_edge_index[1], data_edge_index[1], call_edge_index[1]]
    dst_list = [control_edge_index[1], data_edge_index[1], call_edge_index[1],
                control_edge_index[0], data_edge_index[0], call_edge_index[0]]
    src_idx = jnp.concatenate(
        [s.astype(i32) + e * N for e, s in enumerate(src_list)]
        + [jnp.zeros((E_PAD - E,), i32)]).reshape(NW, GROUPS, SB, K)
    dst_idx = jnp.concatenate(
        [d.astype(i32) for d in dst_list]
        + [jnp.full((E_PAD - E,), N, i32)]).reshape(NW, GROUPS, SB, K)
    idx_all = jnp.stack([src_idx, dst_idx], axis=3)
    zeros_rows = jnp.zeros((RPS, D), f32)

    h0 = _emb_gather(embed, ids_pad)[:N]
    h = h0
    for step in range(6):
        if step < 3:
            Wl, bl, gW, gU, gb = mp1_W, mp1_b, mp1_gru_W, mp1_gru_U, mp1_gru_b
        else:
            Wl, bl, gW, gU, gb = mp2_W, mp2_b, mp2_gru_W, mp2_gru_U, mp2_gru_b
        hw = _hw_matmul(h, Wl, bl.reshape(NUM_ET, 1, D))
        hw_flat = hw.reshape(NUM_ET * N, D)
        parts = _agg_scatter(hw_flat, idx_all, zeros_rows)
        parts3 = parts.reshape(NC, NPAD, D)
        h = _gru_apply(parts3, h, gW, gU, gb)

    aux = jnp.stack([wgsize_log1p, transfer_bytes_log1p], axis=-1)
    gid2d = graph_nodes_list.astype(i32).reshape(1, N)
    return _readout(h, h0, gid2d, Wf, bf.reshape(1, C), Wg, bg.reshape(1, C),
                    aux, W1, b1.reshape(1, GX), W2, b2.reshape(1, C))


# speedup vs baseline: 1.7567x; 1.7567x over previous
import functools

import jax
import jax.numpy as jnp
from jax import lax
from jax.experimental import pallas as pl
from jax.experimental.pallas import tpu as pltpu
from jax.experimental.pallas import tpu_sc as plsc

N = 10000
V = 2230
D = 128
G = 32
C = 2
GX = 64
EC, ED, EK = 160000, 120000, 40000
E = 2 * (EC + ED + EK)
NUM_ET = 6

NC, NS = 2, 16
NW = NC * NS
K = 128
GROUPS = -(-E // (NW * K))
E_PAD = NW * GROUPS * K
EMB_CH = 3
IDS_PAD = NW * EMB_CH * K
NPAD = 10112
RPS = NPAD // NS

BS = 1000
NB = N // BS

f32 = jnp.float32
i32 = jnp.int32

_SC_MESH = dict(core_axis_name="c", subcore_axis_name="s")


def _emb_body(emb_hbm, ids_hbm, out_hbm, idx_v, rows_v, sem):
    c = lax.axis_index("c")
    s = lax.axis_index("s")
    w = s * NC + c
    for j in range(EMB_CH):
        base = (w * EMB_CH + j) * K
        pltpu.sync_copy(ids_hbm.at[pl.ds(base, K)], idx_v)
        pltpu.async_copy(emb_hbm.at[idx_v], rows_v, sem).wait()
        pltpu.sync_copy(rows_v, out_hbm.at[pl.ds(base, K)])


def _emb_gather(embed, ids_pad):
    fn = functools.partial(
        pl.kernel,
        out_type=jax.ShapeDtypeStruct((IDS_PAD, D), f32),
        mesh=plsc.VectorSubcoreMesh(**_SC_MESH),
        scratch_types=[
            pltpu.VMEM((K,), i32),
            pltpu.VMEM((K, D), f32),
            pltpu.SemaphoreType.DMA,
        ],
    )(_emb_body)
    return fn(embed, ids_pad)


def _agg_body(hw_hbm, idx_hbm, zeros_hbm, out_hbm,
              idxv, rows, agg_sh, sem):
    c = lax.axis_index("c")
    s = lax.axis_index("s")
    w = s * NC + c
    pltpu.sync_copy(zeros_hbm, agg_sh.at[pl.ds(s * RPS, RPS)])
    plsc.subcore_barrier()

    def group(g, carry):
        pltpu.sync_copy(idx_hbm.at[w, g], idxv)
        pltpu.async_copy(hw_hbm.at[idxv.at[0]], rows, sem).wait()
        pltpu.sync_copy(rows, agg_sh.at[idxv.at[1]], add=True)
        return carry

    lax.fori_loop(0, GROUPS, group, 0)
    plsc.subcore_barrier()
    pltpu.sync_copy(agg_sh.at[pl.ds(s * RPS, RPS)],
                    out_hbm.at[pl.ds(c * NPAD + s * RPS, RPS)])


def _agg_scatter(hw_flat, idx_all, zeros_rows):
    fn = functools.partial(
        pl.kernel,
        out_type=jax.ShapeDtypeStruct((NC * NPAD, D), f32),
        mesh=plsc.VectorSubcoreMesh(**_SC_MESH),
        scratch_types=[
            pltpu.VMEM((2, K), i32),
            pltpu.VMEM((K, D), f32),
            pltpu.VMEM_SHARED((NPAD, D), f32),
            pltpu.SemaphoreType.DMA,
        ],
    )(_agg_body)
    return fn(hw_flat, idx_all, zeros_rows)


def _hw_body(h_ref, w_ref, b_ref, out_ref):
    out_ref[0] = (jnp.dot(h_ref[...], w_ref[0], preferred_element_type=f32)
                  + b_ref[0])


def _hw_matmul(h, Wl, bl):
    return pl.pallas_call(
        _hw_body,
        grid=(NUM_ET, NB),
        in_specs=[
            pl.BlockSpec((BS, D), lambda e, i: (i, 0)),
            pl.BlockSpec((1, D, D), lambda e, i: (e, 0, 0)),
            pl.BlockSpec((1, 1, D), lambda e, i: (e, 0, 0)),
        ],
        out_specs=pl.BlockSpec((1, BS, D), lambda e, i: (e, i, 0)),
        out_shape=jax.ShapeDtypeStruct((NUM_ET, N, D), f32),
    )(h, Wl, bl)


def _gru_body(parts_ref, h_ref, gw_ref, gu_ref, gb_ref, out_ref):
    agg = parts_ref[0] + parts_ref[1]
    h = h_ref[...]
    dot = lambda a, b: jnp.dot(a, b, preferred_element_type=f32)
    z = jax.nn.sigmoid(dot(agg, gw_ref[0]) + dot(h, gu_ref[0]) + gb_ref[0])
    r = jax.nn.sigmoid(dot(agg, gw_ref[1]) + dot(h, gu_ref[1]) + gb_ref[1])
    hh = jnp.tanh(dot(agg, gw_ref[2]) + dot(r * h, gu_ref[2]) + gb_ref[2])
    out_ref[...] = (1.0 - z) * h + z * hh


def _gru_apply(parts3, h, gW, gU, gb):
    return pl.pallas_call(
        _gru_body,
        grid=(NB,),
        in_specs=[
            pl.BlockSpec((NC, BS, D), lambda i: (0, i, 0)),
            pl.BlockSpec((BS, D), lambda i: (i, 0)),
            pl.BlockSpec((3, D, D), lambda i: (0, 0, 0)),
            pl.BlockSpec((3, D, D), lambda i: (0, 0, 0)),
            pl.BlockSpec((3, D), lambda i: (0, 0)),
        ],
        out_specs=pl.BlockSpec((BS, D), lambda i: (i, 0)),
        out_shape=jax.ShapeDtypeStruct((N, D), f32),
    )(parts3, h, gW, gU, gb)


def _readout_body(h_ref, h0_ref, gid_ref, wf_ref, bf_ref, wg_ref, bg_ref,
                  aux_ref, w1_ref, b1_ref, w2_ref, b2_ref, out_ref):
    h = h_ref[...]
    h0 = h0_ref[...]
    dot = lambda a, b: jnp.dot(a, b, preferred_element_type=f32)
    gate = jax.nn.sigmoid(dot(h, wf_ref[:D]) + dot(h0, wf_ref[D:]) + bf_ref[0])
    val = dot(h, wg_ref[...]) + bg_ref[0]
    gv = gate * val
    onehot = (lax.broadcasted_iota(i32, (G, N), 0) == gid_ref[...]).astype(f32)
    feats = dot(onehot, gv)
    aux = aux_ref[...]
    x = (feats[:, 0:1] * w1_ref[0:1, :] + feats[:, 1:2] * w1_ref[1:2, :]
         + aux[:, 0:1] * w1_ref[2:3, :] + aux[:, 1:2] * w1_ref[3:4, :]
         + b1_ref[...])
    x = jnp.maximum(x, 0.0)
    out_ref[...] = dot(x, w2_ref[...]) + b2_ref[...]


def _readout(h, h0, gid2d, Wf, bf2, Wg, bg2, aux, W1, b12, W2, b22):
    return pl.pallas_call(
        _readout_body,
        out_shape=jax.ShapeDtypeStruct((G, C), f32),
    )(h, h0, gid2d, Wf, bf2, Wg, bg2, aux, W1, b12, W2, b22)


def kernel(node_vocab_ids, control_edge_index, data_edge_index, call_edge_index,
           graph_nodes_list, wgsize_log1p, transfer_bytes_log1p,
           embed, mp1_W, mp1_b, mp1_gru_W, mp1_gru_U, mp1_gru_b,
           mp2_W, mp2_b, mp2_gru_W, mp2_gru_U, mp2_gru_b,
           Wf, bf, Wg, bg, W1, b1, W2, b2):
    ids_pad = jnp.concatenate(
        [node_vocab_ids.astype(i32), jnp.zeros((IDS_PAD - N,), i32)])
    src_list = [control_edge_index[0], data_edge_index[0], call_edge_index[0],
                control_edge_index[1], data_edge_index[1], call_edge_index[1]]
    dst_list = [control_edge_index[1], data_edge_index[1], call_edge_index[1],
                control_edge_index[0], data_edge_index[0], call_edge_index[0]]
    src_idx = jnp.concatenate(
        [s.astype(i32) + e * N for e, s in enumerate(src_list)]
        + [jnp.zeros((E_PAD - E,), i32)]).reshape(NW, GROUPS, K)
    dst_idx = jnp.concatenate(
        [d.astype(i32) for d in dst_list]
        + [jnp.full((E_PAD - E,), N, i32)]).reshape(NW, GROUPS, K)
    idx_all = jnp.stack([src_idx, dst_idx], axis=2)
    zeros_rows = jnp.zeros((RPS, D), f32)

    h0 = _emb_gather(embed, ids_pad)[:N]
    h = h0
    for step in range(6):
        if step < 3:
            Wl, bl, gW, gU, gb = mp1_W, mp1_b, mp1_gru_W, mp1_gru_U, mp1_gru_b
        else:
            Wl, bl, gW, gU, gb = mp2_W, mp2_b, mp2_gru_W, mp2_gru_U, mp2_gru_b
        hw = _hw_matmul(h, Wl, bl.reshape(NUM_ET, 1, D))
        hw_flat = hw.reshape(NUM_ET * N, D)
        parts = _agg_scatter(hw_flat, idx_all, zeros_rows)
        parts3 = parts.reshape(NC, NPAD, D)
        h = _gru_apply(parts3, h, gW, gU, gb)

    aux = jnp.stack([wgsize_log1p, transfer_bytes_log1p], axis=-1)
    gid2d = graph_nodes_list.astype(i32).reshape(1, N)
    return _readout(h, h0, gid2d, Wf, bf.reshape(1, C), Wg, bg.reshape(1, C),
                    aux, W1, b1.reshape(1, GX), W2, b2.reshape(1, C))
